# Initial kernel scaffold; baseline (speedup 1.0000x reference)
import jax
import jax.numpy as jnp
from jax.experimental import pallas as pl


def kernel(x, edge_index, edge_attributes, W1, b1, W2, b2, Wp, bp):
    raise NotImplementedError("write your pallas kernel here")


# speedup vs baseline: 16.1116x; 16.1116x over previous
import functools

import jax
import jax.numpy as jnp
from jax import lax
from jax.experimental import pallas as pl
from jax.experimental.pallas import tpu as pltpu
from jax.experimental.pallas import tpu_sc as plsc

N = 8192
E = 131072
F = 128
K = 16
NC = 2
NS = 16
NW = NC * NS
EW = E // NW
GP = 128
GW = EW // GP
RPT = N // NS

_MESH = dict(core_axis_name="c", subcore_axis_name="s")


def _sc_mesh():
    return plsc.VectorSubcoreMesh(**_MESH)


def _bf16_round(v):
    u = lax.bitcast_convert_type(v, jnp.int32)
    lsb = lax.shift_right_logical(u, 16) & 1
    r = (u + 0x7FFF + lsb) & jnp.int32(-65536)
    return lax.bitcast_convert_type(r, jnp.float32)


@functools.partial(
    pl.kernel,
    out_type=(
        jax.ShapeDtypeStruct((NC, N), jnp.float32),
        jax.ShapeDtypeStruct((NC, N), jnp.float32),
        jax.ShapeDtypeStruct((E // GP, GP), jnp.float32),
    ),
    mesh=_sc_mesh(),
    scratch_types=(
        pltpu.VMEM((GW, GP), jnp.int32),
        pltpu.VMEM((GW, GP), jnp.int32),
        pltpu.VMEM((GW, GP), jnp.float32),
        pltpu.VMEM((GW, GP), jnp.float32),
        pltpu.VMEM((RPT,), jnp.float32),
        pltpu.VMEM_SHARED((N,), jnp.float32),
        pltpu.VMEM_SHARED((N,), jnp.float32),
    ),
)
def _sc_degrees(row_hbm, col_hbm, ea_hbm, degc_out, degr_out, earnd_out,
                ridx, cidx, eav, earv, zbuf, degc_acc, degr_acc):
    cid = lax.axis_index("c")
    sid = lax.axis_index("s")
    wid = cid * NS + sid

    zv = jnp.zeros((16,), jnp.float32)

    def zero_body(i, carry):
        zbuf[pl.ds(i * 16, 16)] = zv
        return carry

    lax.fori_loop(0, RPT // 16, zero_body, 0)
    pltpu.sync_copy(zbuf, degc_acc.at[pl.ds(sid * RPT, RPT)])
    pltpu.sync_copy(zbuf, degr_acc.at[pl.ds(sid * RPT, RPT)])
    plsc.subcore_barrier()

    pltpu.sync_copy(row_hbm.at[pl.ds(wid * GW, GW)], ridx)
    pltpu.sync_copy(col_hbm.at[pl.ds(wid * GW, GW)], cidx)
    pltpu.sync_copy(ea_hbm.at[pl.ds(wid * GW, GW)], eav)

    def grp(j, carry):
        pltpu.sync_copy(eav.at[j], degc_acc.at[cidx.at[j]], add=True)
        pltpu.sync_copy(eav.at[j], degr_acc.at[ridx.at[j]], add=True)
        return carry

    lax.fori_loop(0, GW, grp, 0)

    def rnd(i, carry):
        g = i // (GP // 16)
        o = (i % (GP // 16)) * 16
        earv[g, pl.ds(o, 16)] = _bf16_round(eav[g, pl.ds(o, 16)])
        return carry

    lax.fori_loop(0, GW * (GP // 16), rnd, 0)
    pltpu.sync_copy(earv, earnd_out.at[pl.ds(wid * GW, GW)])
    plsc.subcore_barrier()

    pltpu.sync_copy(degc_acc.at[pl.ds(sid * RPT, RPT)],
                    degc_out.at[cid, pl.ds(sid * RPT, RPT)])
    pltpu.sync_copy(degr_acc.at[pl.ds(sid * RPT, RPT)],
                    degr_out.at[cid, pl.ds(sid * RPT, RPT)])


def _make_sc_scatter(width):
    qn = width // 16

    @functools.partial(
        pl.kernel,
        out_type=jax.ShapeDtypeStruct((NC, N, width), jnp.float32),
        mesh=_sc_mesh(),
        scratch_types=(
            pltpu.VMEM((GW, GP), jnp.int32),
            pltpu.VMEM((GW, GP), jnp.int32),
            pltpu.VMEM((GW, GP), jnp.float32),
            pltpu.VMEM((GP, width), jnp.float32),
            pltpu.VMEM_SHARED((N, width), jnp.float32),
            pltpu.SemaphoreType.DMA,
        ),
        compiler_params=pltpu.CompilerParams(use_tc_tiling_on_sc=(width % 128 == 0)),
    )
    def _sc_scatter(gidx_hbm, sidx_hbm, ea_hbm, y_hbm, acc_out,
                    gidx, sidx, eav, gbuf, acc, sem):
        cid = lax.axis_index("c")
        sid = lax.axis_index("s")
        wid = cid * NS + sid

        zv = jnp.zeros((16,), jnp.float32)

        def zero_row(e, carry):
            for q in range(qn):
                gbuf[e, pl.ds(q * 16, 16)] = zv
            return carry

        lax.fori_loop(0, GP, zero_row, 0)
        for t in range(RPT // GP):
            pltpu.sync_copy(gbuf, acc.at[pl.ds(sid * RPT + t * GP, GP)])
        plsc.subcore_barrier()

        pltpu.sync_copy(gidx_hbm.at[pl.ds(wid * GW, GW)], gidx)
        pltpu.sync_copy(sidx_hbm.at[pl.ds(wid * GW, GW)], sidx)
        pltpu.sync_copy(ea_hbm.at[pl.ds(wid * GW, GW)], eav)

        def grp(j, carry):
            pltpu.async_copy(y_hbm.at[gidx.at[j]], gbuf, sem).wait()

            def scale(g, c2):
                wv = eav[j, pl.ds(g * 16, 16)]
                for l in range(16):
                    w = wv[l]
                    e = g * 16 + l
                    for q in range(qn):
                        gbuf[e, pl.ds(q * 16, 16)] = gbuf[e, pl.ds(q * 16, 16)] * w
                return c2

            lax.fori_loop(0, GP // 16, scale, 0)
            pltpu.sync_copy(gbuf, acc.at[sidx.at[j]], add=True)
            return carry

        lax.fori_loop(0, GW, grp, 0)
        plsc.subcore_barrier()

        for t in range(RPT // GP):
            pltpu.sync_copy(acc.at[pl.ds(sid * RPT + t * GP, GP)],
                            acc_out.at[cid, pl.ds(sid * RPT + t * GP, GP)])

    return _sc_scatter


_sc_scatter_f = _make_sc_scatter(F)
_sc_scatter_k = _make_sc_scatter(K)


_BN = 1024


def _tc1_body(x_ref, w_ref, a0_ref, a1_ref, y_ref, dis_ref):
    dis = lax.rsqrt(a0_ref[...] + a1_ref[...] + 1.0)
    xw = jnp.dot(x_ref[...], w_ref[...], preferred_element_type=jnp.float32)
    y_ref[...] = dis * xw
    dis_ref[...] = dis


def _tc1(x, w1, a0, a1):
    return pl.pallas_call(
        _tc1_body,
        grid=(N // _BN,),
        in_specs=[
            pl.BlockSpec((_BN, F), lambda i: (i, 0)),
            pl.BlockSpec((F, F), lambda i: (0, 0)),
            pl.BlockSpec((_BN, 1), lambda i: (i, 0)),
            pl.BlockSpec((_BN, 1), lambda i: (i, 0)),
        ],
        out_specs=[
            pl.BlockSpec((_BN, F), lambda i: (i, 0)),
            pl.BlockSpec((_BN, 1), lambda i: (i, 0)),
        ],
        out_shape=[
            jax.ShapeDtypeStruct((N, F), jnp.float32),
            jax.ShapeDtypeStruct((N, 1), jnp.float32),
        ],
    )(x, w1, a0, a1)


def _tc2_body(p0_ref, p1_ref, y_ref, dis_ref, b_ref, w_ref, y2_ref):
    dis = dis_ref[...]
    h = jnp.maximum(dis * (p0_ref[...] + p1_ref[...] + y_ref[...]) + b_ref[...], 0.0)
    y2_ref[...] = dis * jnp.dot(h, w_ref[...], preferred_element_type=jnp.float32)


def _tc2(p0, p1, y, dis, b, w):
    return pl.pallas_call(
        _tc2_body,
        grid=(N // _BN,),
        in_specs=[
            pl.BlockSpec((_BN, F), lambda i: (i, 0)),
            pl.BlockSpec((_BN, F), lambda i: (i, 0)),
            pl.BlockSpec((_BN, F), lambda i: (i, 0)),
            pl.BlockSpec((_BN, 1), lambda i: (i, 0)),
            pl.BlockSpec((1, F), lambda i: (0, 0)),
            pl.BlockSpec((F, F), lambda i: (0, 0)),
        ],
        out_specs=pl.BlockSpec((_BN, F), lambda i: (i, 0)),
        out_shape=jax.ShapeDtypeStruct((N, F), jnp.float32),
    )(p0, p1, y, dis, b, w)


def _tc3_body(q0_ref, q1_ref, y2_ref, dis_ref, b_ref, wp_ref, bp_ref,
              s_ref, srnd_ref):
    dis = dis_ref[...]
    h = jnp.maximum(dis * (q0_ref[...] + q1_ref[...] + y2_ref[...]) + b_ref[...], 0.0)
    logits = jnp.dot(h, wp_ref[...], preferred_element_type=jnp.float32) + bp_ref[...]
    mx = jnp.max(logits, axis=-1, keepdims=True)
    ex = jnp.exp(logits - mx)
    s = ex / jnp.sum(ex, axis=-1, keepdims=True)
    s_ref[...] = s
    srnd_ref[...] = _bf16_round(s)


def _tc3(q0, q1, y2, dis, b, wp, bp):
    return pl.pallas_call(
        _tc3_body,
        grid=(N // _BN,),
        in_specs=[
            pl.BlockSpec((_BN, F), lambda i: (i, 0)),
            pl.BlockSpec((_BN, F), lambda i: (i, 0)),
            pl.BlockSpec((_BN, F), lambda i: (i, 0)),
            pl.BlockSpec((_BN, 1), lambda i: (i, 0)),
            pl.BlockSpec((1, F), lambda i: (0, 0)),
            pl.BlockSpec((F, K), lambda i: (0, 0)),
            pl.BlockSpec((1, K), lambda i: (0, 0)),
        ],
        out_specs=[
            pl.BlockSpec((_BN, K), lambda i: (i, 0)),
            pl.BlockSpec((_BN, K), lambda i: (i, 0)),
        ],
        out_shape=[
            jax.ShapeDtypeStruct((N, K), jnp.float32),
            jax.ShapeDtypeStruct((N, K), jnp.float32),
        ],
    )(q0, q1, y2, dis, b, wp, bp)


def _tc4_body(s_ref, srnd_ref, as0_ref, as1_ref, d0_ref, d1_ref,
              loss_ref, spec_ref, clus_ref):
    s = s_ref[...]
    s_b = srnd_ref[...].astype(jnp.bfloat16)
    ats = as0_ref[...] + as1_ref[...]
    ats_b = _bf16_round(ats).astype(jnp.bfloat16)
    out_adj = lax.dot_general(ats_b, s_b, (((0,), (0,)), ((), ())),
                              preferred_element_type=jnp.float32)
    degr = d0_ref[...] + d1_ref[...]
    m = jnp.sum(degr) / 2.0
    degr_b = _bf16_round(degr).astype(jnp.bfloat16)
    sdeg = lax.dot_general(s_b, degr_b, (((0,), (0,)), ((), ())),
                           preferred_element_type=jnp.float32)
    tn = (sdeg * sdeg) * 0.5 / m
    ia = lax.broadcasted_iota(jnp.int32, (K, K), 0)
    ib = lax.broadcasted_iota(jnp.int32, (K, K), 1)
    to = jnp.sum(jnp.where(ia == ib, out_adj, 0.0), axis=1, keepdims=True)
    trace = jnp.sum(to - tn)
    spectral = -trace / 2.0 / m
    cs = jnp.sum(s, axis=0)
    cl = jnp.sqrt(jnp.sum(cs * cs))
    cluster = cl / float(N) * jnp.sqrt(float(K)) - 1.0
    loss_ref[...] = jnp.broadcast_to(100.0 * (spectral + cluster), (1, 1))
    spec_ref[...] = jnp.broadcast_to(100.0 * spectral, (1, 1))
    clus_ref[...] = jnp.broadcast_to(100.0 * cluster, (1, 1))


def _tc4(s, srnd, as0, as1, d0, d1):
    return pl.pallas_call(
        _tc4_body,
        out_shape=[
            jax.ShapeDtypeStruct((1, 1), jnp.float32),
            jax.ShapeDtypeStruct((1, 1), jnp.float32),
            jax.ShapeDtypeStruct((1, 1), jnp.float32),
        ],
    )(s, srnd, as0, as1, d0, d1)


def kernel(x, edge_index, edge_attributes, W1, b1, W2, b2, Wp, bp):
    row = edge_index[0].reshape(E // GP, GP)
    col = edge_index[1].reshape(E // GP, GP)
    ea2 = edge_attributes.reshape(E // GP, GP)

    degc_p, degr_p, ea_rnd = _sc_degrees(row, col, ea2)

    a0 = degc_p[0].reshape(N, 1)
    a1 = degc_p[1].reshape(N, 1)
    y1, dis = _tc1(x, W1, a0, a1)

    acc1 = _sc_scatter_f(row, col, ea2, y1)
    y2 = _tc2(acc1[0], acc1[1], y1, dis, b1.reshape(1, F), W2)

    acc2 = _sc_scatter_f(row, col, ea2, y2)
    s, s_rnd = _tc3(acc2[0], acc2[1], y2, dis, b2.reshape(1, F), Wp, bp.reshape(1, K))

    asum = _sc_scatter_k(row, col, ea_rnd, s_rnd)

    d0 = degr_p[0].reshape(N, 1)
    d1 = degr_p[1].reshape(N, 1)
    loss, spec, clus = _tc4(s, s_rnd, asum[0], asum[1], d0, d1)

    return (s.reshape(1, N, K), loss[0, 0], spec[0, 0], clus[0, 0])


# speedup vs baseline: 20.7413x; 1.2873x over previous
import functools

import jax
import jax.numpy as jnp
from jax import lax
from jax.experimental import pallas as pl
from jax.experimental.pallas import tpu as pltpu
from jax.experimental.pallas import tpu_sc as plsc

N = 8192
E = 131072
F = 128
K = 16
NC = 2
NS = 16
NW = NC * NS
EW = E // NW
GP = 128
GW = EW // GP
RPT = N // NS

_MESH = dict(core_axis_name="c", subcore_axis_name="s")


def _sc_mesh():
    return plsc.VectorSubcoreMesh(**_MESH)


def _bf16_round(v):
    u = lax.bitcast_convert_type(v, jnp.int32)
    lsb = lax.shift_right_logical(u, 16) & 1
    r = (u + 0x7FFF + lsb) & jnp.int32(-65536)
    return lax.bitcast_convert_type(r, jnp.float32)


@functools.partial(
    pl.kernel,
    out_type=(
        jax.ShapeDtypeStruct((NC, N), jnp.float32),
        jax.ShapeDtypeStruct((NC, N), jnp.float32),
        jax.ShapeDtypeStruct((E // GP, GP), jnp.float32),
    ),
    mesh=_sc_mesh(),
    scratch_types=(
        pltpu.VMEM((GW, GP), jnp.int32),
        pltpu.VMEM((GW, GP), jnp.int32),
        pltpu.VMEM((GW, GP), jnp.float32),
        pltpu.VMEM((GW, GP), jnp.float32),
        pltpu.VMEM((RPT,), jnp.float32),
        pltpu.VMEM_SHARED((N,), jnp.float32),
        pltpu.VMEM_SHARED((N,), jnp.float32),
    ),
)
def _sc_degrees(row_hbm, col_hbm, ea_hbm, degc_out, degr_out, earnd_out,
                ridx, cidx, eav, earv, zbuf, degc_acc, degr_acc):
    cid = lax.axis_index("c")
    sid = lax.axis_index("s")
    wid = cid * NS + sid

    zv = jnp.zeros((16,), jnp.float32)

    def zero_body(i, carry):
        zbuf[pl.ds(i * 16, 16)] = zv
        return carry

    lax.fori_loop(0, RPT // 16, zero_body, 0)
    pltpu.sync_copy(zbuf, degc_acc.at[pl.ds(sid * RPT, RPT)])
    pltpu.sync_copy(zbuf, degr_acc.at[pl.ds(sid * RPT, RPT)])
    plsc.subcore_barrier()

    pltpu.sync_copy(row_hbm.at[pl.ds(wid * GW, GW)], ridx)
    pltpu.sync_copy(col_hbm.at[pl.ds(wid * GW, GW)], cidx)
    pltpu.sync_copy(ea_hbm.at[pl.ds(wid * GW, GW)], eav)

    def grp(j, carry):
        pltpu.sync_copy(eav.at[j], degc_acc.at[cidx.at[j]], add=True)
        pltpu.sync_copy(eav.at[j], degr_acc.at[ridx.at[j]], add=True)
        return carry

    lax.fori_loop(0, GW, grp, 0)

    def rnd(i, carry):
        g = i // (GP // 16)
        o = (i % (GP // 16)) * 16
        earv[g, pl.ds(o, 16)] = _bf16_round(eav[g, pl.ds(o, 16)])
        return carry

    lax.fori_loop(0, GW * (GP // 16), rnd, 0)
    pltpu.sync_copy(earv, earnd_out.at[pl.ds(wid * GW, GW)])
    plsc.subcore_barrier()

    pltpu.sync_copy(degc_acc.at[pl.ds(sid * RPT, RPT)],
                    degc_out.at[cid, pl.ds(sid * RPT, RPT)])
    pltpu.sync_copy(degr_acc.at[pl.ds(sid * RPT, RPT)],
                    degr_out.at[cid, pl.ds(sid * RPT, RPT)])


def _make_sc_scatter(width):
    qn = width // 16

    @functools.partial(
        pl.kernel,
        out_type=jax.ShapeDtypeStruct((NC, N, width), jnp.float32),
        mesh=_sc_mesh(),
        scratch_types=(
            pltpu.VMEM((GW, GP), jnp.int32),
            pltpu.VMEM((GW, GP), jnp.int32),
            pltpu.VMEM((GW, GP), jnp.float32),
            pltpu.VMEM((GP, width), jnp.float32),
            pltpu.VMEM((GP, width), jnp.float32),
            pltpu.VMEM_SHARED((N, width), jnp.float32),
            pltpu.SemaphoreType.DMA,
            pltpu.SemaphoreType.DMA,
            pltpu.SemaphoreType.DMA,
            pltpu.SemaphoreType.DMA,
        ),
        compiler_params=pltpu.CompilerParams(use_tc_tiling_on_sc=(width % 128 == 0)),
    )
    def _sc_scatter(gidx_hbm, sidx_hbm, ea_hbm, y_hbm, acc_out,
                    gidx, sidx, eav, bufa, bufb, acc, ga, gb, sa, sb):
        cid = lax.axis_index("c")
        sid = lax.axis_index("s")
        wid = cid * NS + sid

        zv = jnp.zeros((16,), jnp.float32)

        def zero_row(e, carry):
            for q in range(qn):
                bufa[e, pl.ds(q * 16, 16)] = zv
            return carry

        lax.fori_loop(0, GP, zero_row, 0)
        for t in range(RPT // GP):
            pltpu.sync_copy(bufa, acc.at[pl.ds(sid * RPT + t * GP, GP)])
        plsc.subcore_barrier()

        pltpu.sync_copy(gidx_hbm.at[pl.ds(wid * GW, GW)], gidx)
        pltpu.sync_copy(sidx_hbm.at[pl.ds(wid * GW, GW)], sidx)
        pltpu.sync_copy(ea_hbm.at[pl.ds(wid * GW, GW)], eav)

        def scale(buf, j):
            def body(g, c2):
                wv = eav[j, pl.ds(g * 16, 16)]
                for l in range(16):
                    w = wv[l]
                    e = g * 16 + l
                    for q in range(qn):
                        buf[e, pl.ds(q * 16, 16)] = buf[e, pl.ds(q * 16, 16)] * w
                return c2

            lax.fori_loop(0, GP // 16, body, 0)

        pltpu.async_copy(y_hbm.at[gidx.at[0]], bufa, ga)

        def pair(k, carry):
            a = 2 * k
            b = 2 * k + 1
            pltpu.make_async_copy(y_hbm.at[gidx.at[a]], bufa, ga).wait()

            @pl.when(k > 0)
            def _():
                pltpu.make_async_copy(bufb, acc.at[sidx.at[b - 2]], sb).wait()

            pltpu.async_copy(y_hbm.at[gidx.at[b]], bufb, gb)
            scale(bufa, a)
            d_sa = pltpu.async_copy(bufa, acc.at[sidx.at[a]], sa, add=True)
            pltpu.make_async_copy(y_hbm.at[gidx.at[b]], bufb, gb).wait()
            d_sa.wait()

            @pl.when(k < GW // 2 - 1)
            def _():
                pltpu.async_copy(y_hbm.at[gidx.at[a + 2]], bufa, ga)

            scale(bufb, b)
            pltpu.async_copy(bufb, acc.at[sidx.at[b]], sb, add=True)
            return carry

        lax.fori_loop(0, GW // 2, pair, 0)
        pltpu.make_async_copy(bufb, acc.at[sidx.at[GW - 1]], sb).wait()
        plsc.subcore_barrier()

        for t in range(RPT // GP):
            pltpu.sync_copy(acc.at[pl.ds(sid * RPT + t * GP, GP)],
                            acc_out.at[cid, pl.ds(sid * RPT + t * GP, GP)])

    return _sc_scatter


_sc_scatter_f = _make_sc_scatter(F)
_sc_scatter_k = _make_sc_scatter(K)


_BN = 1024


def _tc1_body(x_ref, w_ref, a0_ref, a1_ref, y_ref, dis_ref):
    dis = lax.rsqrt(a0_ref[...] + a1_ref[...] + 1.0)
    xw = jnp.dot(x_ref[...], w_ref[...], preferred_element_type=jnp.float32)
    y_ref[...] = dis * xw
    dis_ref[...] = dis


def _tc1(x, w1, a0, a1):
    return pl.pallas_call(
        _tc1_body,
        grid=(N // _BN,),
        in_specs=[
            pl.BlockSpec((_BN, F), lambda i: (i, 0)),
            pl.BlockSpec((F, F), lambda i: (0, 0)),
            pl.BlockSpec((_BN, 1), lambda i: (i, 0)),
            pl.BlockSpec((_BN, 1), lambda i: (i, 0)),
        ],
        out_specs=[
            pl.BlockSpec((_BN, F), lambda i: (i, 0)),
            pl.BlockSpec((_BN, 1), lambda i: (i, 0)),
        ],
        out_shape=[
            jax.ShapeDtypeStruct((N, F), jnp.float32),
            jax.ShapeDtypeStruct((N, 1), jnp.float32),
        ],
    )(x, w1, a0, a1)


def _tc2_body(p0_ref, p1_ref, y_ref, dis_ref, b_ref, w_ref, y2_ref):
    dis = dis_ref[...]
    h = jnp.maximum(dis * (p0_ref[...] + p1_ref[...] + y_ref[...]) + b_ref[...], 0.0)
    y2_ref[...] = dis * jnp.dot(h, w_ref[...], preferred_element_type=jnp.float32)


def _tc2(p0, p1, y, dis, b, w):
    return pl.pallas_call(
        _tc2_body,
        grid=(N // _BN,),
        in_specs=[
            pl.BlockSpec((_BN, F), lambda i: (i, 0)),
            pl.BlockSpec((_BN, F), lambda i: (i, 0)),
            pl.BlockSpec((_BN, F), lambda i: (i, 0)),
            pl.BlockSpec((_BN, 1), lambda i: (i, 0)),
            pl.BlockSpec((1, F), lambda i: (0, 0)),
            pl.BlockSpec((F, F), lambda i: (0, 0)),
        ],
        out_specs=pl.BlockSpec((_BN, F), lambda i: (i, 0)),
        out_shape=jax.ShapeDtypeStruct((N, F), jnp.float32),
    )(p0, p1, y, dis, b, w)


def _tc3_body(q0_ref, q1_ref, y2_ref, dis_ref, b_ref, wp_ref, bp_ref,
              s_ref, srnd_ref):
    dis = dis_ref[...]
    h = jnp.maximum(dis * (q0_ref[...] + q1_ref[...] + y2_ref[...]) + b_ref[...], 0.0)
    logits = jnp.dot(h, wp_ref[...], preferred_element_type=jnp.float32) + bp_ref[...]
    mx = jnp.max(logits, axis=-1, keepdims=True)
    ex = jnp.exp(logits - mx)
    s = ex / jnp.sum(ex, axis=-1, keepdims=True)
    s_ref[...] = s
    srnd_ref[...] = _bf16_round(s)


def _tc3(q0, q1, y2, dis, b, wp, bp):
    return pl.pallas_call(
        _tc3_body,
        grid=(N // _BN,),
        in_specs=[
            pl.BlockSpec((_BN, F), lambda i: (i, 0)),
            pl.BlockSpec((_BN, F), lambda i: (i, 0)),
            pl.BlockSpec((_BN, F), lambda i: (i, 0)),
            pl.BlockSpec((_BN, 1), lambda i: (i, 0)),
            pl.BlockSpec((1, F), lambda i: (0, 0)),
            pl.BlockSpec((F, K), lambda i: (0, 0)),
            pl.BlockSpec((1, K), lambda i: (0, 0)),
        ],
        out_specs=[
            pl.BlockSpec((_BN, K), lambda i: (i, 0)),
            pl.BlockSpec((_BN, K), lambda i: (i, 0)),
        ],
        out_shape=[
            jax.ShapeDtypeStruct((N, K), jnp.float32),
            jax.ShapeDtypeStruct((N, K), jnp.float32),
        ],
    )(q0, q1, y2, dis, b, wp, bp)


def _tc4_body(s_ref, srnd_ref, as0_ref, as1_ref, d0_ref, d1_ref,
              loss_ref, spec_ref, clus_ref):
    s = s_ref[...]
    s_b = srnd_ref[...].astype(jnp.bfloat16)
    ats = as0_ref[...] + as1_ref[...]
    ats_b = _bf16_round(ats).astype(jnp.bfloat16)
    out_adj = lax.dot_general(ats_b, s_b, (((0,), (0,)), ((), ())),
                              preferred_element_type=jnp.float32)
    degr = d0_ref[...] + d1_ref[...]
    m = jnp.sum(degr) / 2.0
    degr_b = _bf16_round(degr).astype(jnp.bfloat16)
    sdeg = lax.dot_general(s_b, degr_b, (((0,), (0,)), ((), ())),
                           preferred_element_type=jnp.float32)
    tn = (sdeg * sdeg) * 0.5 / m
    ia = lax.broadcasted_iota(jnp.int32, (K, K), 0)
    ib = lax.broadcasted_iota(jnp.int32, (K, K), 1)
    to = jnp.sum(jnp.where(ia == ib, out_adj, 0.0), axis=1, keepdims=True)
    trace = jnp.sum(to - tn)
    spectral = -trace / 2.0 / m
    cs = jnp.sum(s, axis=0)
    cl = jnp.sqrt(jnp.sum(cs * cs))
    cluster = cl / float(N) * jnp.sqrt(float(K)) - 1.0
    loss_ref[...] = jnp.broadcast_to(100.0 * (spectral + cluster), (1, 1))
    spec_ref[...] = jnp.broadcast_to(100.0 * spectral, (1, 1))
    clus_ref[...] = jnp.broadcast_to(100.0 * cluster, (1, 1))


def _tc4(s, srnd, as0, as1, d0, d1):
    return pl.pallas_call(
        _tc4_body,
        out_shape=[
            jax.ShapeDtypeStruct((1, 1), jnp.float32),
            jax.ShapeDtypeStruct((1, 1), jnp.float32),
            jax.ShapeDtypeStruct((1, 1), jnp.float32),
        ],
    )(s, srnd, as0, as1, d0, d1)


def kernel(x, edge_index, edge_attributes, W1, b1, W2, b2, Wp, bp):
    row = edge_index[0].reshape(E // GP, GP)
    col = edge_index[1].reshape(E // GP, GP)
    ea2 = edge_attributes.reshape(E // GP, GP)

    degc_p, degr_p, ea_rnd = _sc_degrees(row, col, ea2)

    a0 = degc_p[0].reshape(N, 1)
    a1 = degc_p[1].reshape(N, 1)
    y1, dis = _tc1(x, W1, a0, a1)

    acc1 = _sc_scatter_f(row, col, ea2, y1)
    y2 = _tc2(acc1[0], acc1[1], y1, dis, b1.reshape(1, F), W2)

    acc2 = _sc_scatter_f(row, col, ea2, y2)
    s, s_rnd = _tc3(acc2[0], acc2[1], y2, dis, b2.reshape(1, F), Wp, bp.reshape(1, K))

    asum = _sc_scatter_k(row, col, ea_rnd, s_rnd)

    d0 = degr_p[0].reshape(N, 1)
    d1 = degr_p[1].reshape(N, 1)
    loss, spec, clus = _tc4(s, s_rnd, asum[0], asum[1], d0, d1)

    return (s.reshape(1, N, K), loss[0, 0], spec[0, 0], clus[0, 0])
